# Initial kernel scaffold; baseline (speedup 1.0000x reference)
import jax
import jax.numpy as jnp
from jax.experimental import pallas as pl


def kernel(x, table):
    raise NotImplementedError("write your pallas kernel here")


# speedup vs baseline: 4.5688x; 4.5688x over previous
import functools

import jax
import jax.numpy as jnp
import numpy as np
from jax import lax
from jax.experimental import pallas as pl
from jax.experimental.pallas import tpu as pltpu
from jax.experimental.pallas import tpu_sc as plsc

DIM = 32
SCALE = np.float32(np.sqrt(np.float64(DIM)))

KW = 128
KC = 8
CHUNK = KC * KW


def _sc_embed(x2d, table, B):
    info = plsc.get_sparse_core_info()
    num_workers = info.num_cores * info.num_subcores
    b_per_w = B // num_workers
    n_chunks = b_per_w // CHUNK
    mesh = plsc.VectorSubcoreMesh(core_axis_name="c", subcore_axis_name="s")

    @functools.partial(
        pl.kernel,
        mesh=mesh,
        compiler_params=pltpu.CompilerParams(use_tc_tiling_on_sc=False),
        out_type=jax.ShapeDtypeStruct((B, DIM), jnp.float32),
        scratch_types=[
            pltpu.VMEM((KC, KW), jnp.int32),
            pltpu.VMEM((CHUNK, DIM), jnp.float32),
            pltpu.SemaphoreType.DMA,
        ],
    )
    def k(x_hbm, table_hbm, out_hbm, idx_v, rows_v, sem):
        wid = lax.axis_index("s") * info.num_cores + lax.axis_index("c")
        base = wid * b_per_w

        def chunk_body(c, carry):
            off = base + c * CHUNK
            pltpu.sync_copy(
                x_hbm.at[pl.ds(pl.multiple_of(off // KW, KC), KC)], idx_v
            )
            copies = []
            for j in range(KC):
                copies.append(
                    pltpu.async_copy(
                        table_hbm.at[idx_v.at[j]],
                        rows_v.at[pl.ds(j * KW, KW)],
                        sem,
                    )
                )
            for cp in copies:
                cp.wait()

            def scale_body(r, _):
                rows_v[r, pl.ds(0, 16)] = rows_v[r, pl.ds(0, 16)] * SCALE
                rows_v[r, pl.ds(16, 16)] = rows_v[r, pl.ds(16, 16)] * SCALE
                return _

            lax.fori_loop(0, CHUNK, scale_body, 0, unroll=4)

            pltpu.sync_copy(rows_v, out_hbm.at[pl.ds(off, CHUNK)])
            return carry

        lax.fori_loop(0, n_chunks, chunk_body, 0)

    return k(x2d, table)


def kernel(x, table):
    B = x.shape[0] * x.shape[1]
    x2d = x.reshape(B // KW, KW).astype(jnp.int32)
    out = _sc_embed(x2d, table, B)
    return out.reshape(x.shape[0], x.shape[1], DIM)


# speedup vs baseline: 4.9760x; 1.0891x over previous
import functools

import jax
import jax.numpy as jnp
import numpy as np
from jax import lax
from jax.experimental import pallas as pl
from jax.experimental.pallas import tpu as pltpu
from jax.experimental.pallas import tpu_sc as plsc

DIM = 32
SCALE = np.float32(np.sqrt(np.float64(DIM)))

KW = 128
KC = 4
CHUNK = KC * KW
NBUF = 4


def _sc_embed(x2d, table, B):
    info = plsc.get_sparse_core_info()
    num_workers = info.num_cores * info.num_subcores
    b_per_w = B // num_workers
    n_chunks = b_per_w // CHUNK
    n_groups = n_chunks // NBUF
    mesh = plsc.VectorSubcoreMesh(core_axis_name="c", subcore_axis_name="s")

    @functools.partial(
        pl.kernel,
        mesh=mesh,
        compiler_params=pltpu.CompilerParams(use_tc_tiling_on_sc=False),
        out_type=jax.ShapeDtypeStruct((B, DIM), jnp.float32),
        scratch_types=[
            pltpu.VMEM((NBUF, KC, KW), jnp.int32),
            pltpu.VMEM((NBUF, CHUNK, DIM), jnp.float32),
        ]
        + [pltpu.SemaphoreType.DMA] * (2 * NBUF),
    )
    def k(x_hbm, table_hbm, out_hbm, idx_v, rows_v, *sems):
        g_sems, s_sems = sems[:NBUF], sems[NBUF:]
        wid = lax.axis_index("s") * info.num_cores + lax.axis_index("c")
        base = wid * b_per_w

        def fire_gather(ch, b):
            off = base + ch * CHUNK
            pltpu.sync_copy(
                x_hbm.at[pl.ds(pl.multiple_of(off // KW, KC), KC)],
                idx_v.at[b],
            )
            for j in range(KC):
                pltpu.async_copy(
                    table_hbm.at[idx_v.at[b, j]],
                    rows_v.at[b, pl.ds(j * KW, KW)],
                    g_sems[b],
                )

        def wait_gather(b):
            pltpu.make_async_copy(
                out_hbm.at[pl.ds(0, CHUNK)], rows_v.at[b], g_sems[b]
            ).wait()

        def fire_store(ch, b):
            off = base + ch * CHUNK
            pltpu.async_copy(rows_v.at[b], out_hbm.at[pl.ds(off, CHUNK)], s_sems[b])

        def wait_store(b):
            pltpu.make_async_copy(
                out_hbm.at[pl.ds(0, CHUNK)], rows_v.at[b], s_sems[b]
            ).wait()

        def scale(b):
            def scale_body(r, carry):
                rows_v[b, r, pl.ds(0, 16)] = rows_v[b, r, pl.ds(0, 16)] * SCALE
                rows_v[b, r, pl.ds(16, 16)] = rows_v[b, r, pl.ds(16, 16)] * SCALE
                return carry

            lax.fori_loop(0, CHUNK, scale_body, 0, unroll=4)

        for b in range(NBUF - 1):
            fire_gather(b, b)

        def group_body(g, carry):
            for b in range(NBUF):
                c = g * NBUF + b
                bf = (b - 1) % NBUF
                wait_gather(b)
                scale(b)
                fire_store(c, b)
                if b == 0:

                    @pl.when(g > 0)
                    def _():
                        wait_store(bf)

                    fire_gather(c + NBUF - 1, bf)
                else:
                    f = c + NBUF - 1

                    @pl.when(f < n_chunks)
                    def _():
                        wait_store(bf)
                        fire_gather(f, bf)

            return carry

        lax.fori_loop(0, n_groups, group_body, 0)
        for b in range(NBUF):
            wait_store(b)

    return k(x2d, table)


def kernel(x, table):
    B = x.shape[0] * x.shape[1]
    x2d = x.reshape(B // KW, KW).astype(jnp.int32)
    out = _sc_embed(x2d, table, B)
    return out.reshape(x.shape[0], x.shape[1], DIM)
